# baseline (speedup 1.0000x reference)
import functools

import jax
import jax.numpy as jnp
from jax import lax
from jax.experimental import pallas as pl
from jax.experimental.pallas import tpu as pltpu
from jax.experimental.pallas import tpu_sc as plsc

IDX_MINOR = 128
CHUNK = 400
D_PAD = 128
_SLICES = [(0, 128), (128, 128), (256, 128), (384, 16)]


def _gather_call(n_idx):
    info = plsc.get_sparse_core_info()
    nc, ns = info.num_cores, info.num_subcores
    nw = nc * ns
    assert n_idx % (nw * 2 * CHUNK) == 0
    per_w = n_idx // nw
    n_pairs = per_w // (2 * CHUNK)

    mesh = plsc.VectorSubcoreMesh(core_axis_name="c", subcore_axis_name="s")

    @functools.partial(
        pl.kernel,
        mesh=mesh,
        out_type=jax.ShapeDtypeStruct((n_idx, D_PAD), jnp.float32),
        scratch_types=[
            pltpu.VMEM((per_w,), jnp.int32),
            pltpu.VMEM((CHUNK, D_PAD), jnp.float32),
            pltpu.VMEM((CHUNK, D_PAD), jnp.float32),
            pltpu.SemaphoreType.DMA,
            pltpu.SemaphoreType.DMA,
            pltpu.SemaphoreType.DMA,
            pltpu.SemaphoreType.DMA,
        ],
    )
    def run(idx_hbm, table_hbm, out_hbm, idx_v, rows_a, rows_b,
            gsem_a, gsem_b, wsem_a, wsem_b):
        wid = lax.axis_index("s") * nc + lax.axis_index("c")
        base = wid * per_w
        pltpu.sync_copy(idx_hbm.at[pl.ds(pl.multiple_of(base, 16), per_w)],
                        idx_v)

        def fire_gather(c, rows_v, gsem):
            coff = pl.multiple_of(c * CHUNK, 16)
            for (o, n) in _SLICES:
                pltpu.async_copy(
                    table_hbm.at[idx_v.at[pl.ds(coff + o, n)]],
                    rows_v.at[pl.ds(o, n)],
                    gsem)

        def wait_gather(c, rows_v, gsem):
            coff = pl.multiple_of(c * CHUNK, 16)
            for (o, n) in _SLICES:
                pltpu.make_async_copy(
                    table_hbm.at[idx_v.at[pl.ds(coff + o, n)]],
                    rows_v.at[pl.ds(o, n)],
                    gsem).wait()

        def fire_write(c, rows_v, wsem):
            off = pl.multiple_of(base + c * CHUNK, 16)
            pltpu.async_copy(rows_v, out_hbm.at[pl.ds(off, CHUNK)], wsem)

        def wait_write(c, rows_v, wsem):
            off = pl.multiple_of(base + c * CHUNK, 16)
            pltpu.make_async_copy(
                rows_v, out_hbm.at[pl.ds(off, CHUNK)], wsem).wait()

        fire_gather(0, rows_a, gsem_a)

        def body(k, carry):
            c0 = 2 * k
            c1 = c0 + 1
            @pl.when(k > 0)
            def _():
                wait_write(c1 - 2, rows_b, wsem_b)
            fire_gather(c1, rows_b, gsem_b)
            wait_gather(c0, rows_a, gsem_a)
            fire_write(c0, rows_a, wsem_a)
            wait_gather(c1, rows_b, gsem_b)

            @pl.when(k < n_pairs - 1)
            def _():
                wait_write(c0, rows_a, wsem_a)
                fire_gather(c0 + 2, rows_a, gsem_a)
            fire_write(c1, rows_b, wsem_b)
            return carry

        lax.fori_loop(0, n_pairs, body, 0)
        wait_write(2 * n_pairs - 2, rows_a, wsem_a)
        wait_write(2 * n_pairs - 1, rows_b, wsem_b)

    return run


def _pad_transpose_call(n_rows, d):
    blk = 4096

    def body(i_ref, o_ref):
        o_ref[:, :d] = i_ref[...].T
        o_ref[:, d:] = jnp.zeros((blk, D_PAD - d), jnp.float32)

    return pl.pallas_call(
        body,
        grid=(pl.cdiv(n_rows, blk),),
        in_specs=[pl.BlockSpec((d, blk), lambda i: (0, i))],
        out_specs=pl.BlockSpec((blk, D_PAD), lambda i: (i, 0)),
        out_shape=jax.ShapeDtypeStruct((n_rows, D_PAD), jnp.float32),
    )


def kernel(word_ids, embed_table):
    b0, b1 = word_ids.shape
    n_rows, d = embed_table.shape
    n_idx = b0 * b1
    idx1d = word_ids.reshape(-1).astype(jnp.int32)
    table_p = _pad_transpose_call(n_rows, d)(embed_table.T)
    out_pad = _gather_call(n_idx)(idx1d, table_p)
    return out_pad[:, :d].reshape(b0, b1, d)


# speedup vs baseline: 1.0107x; 1.0107x over previous
import functools

import jax
import jax.numpy as jnp
from jax import lax
from jax.experimental import pallas as pl
from jax.experimental.pallas import tpu as pltpu
from jax.experimental.pallas import tpu_sc as plsc

IDX_MINOR = 128
CHUNK = 400
D_PAD = 128
_SLICES = [(0, 128), (128, 128), (256, 128), (384, 16)]


def _gather_call(n_idx):
    info = plsc.get_sparse_core_info()
    nc, ns = info.num_cores, info.num_subcores
    nw = nc * ns
    assert n_idx % (nw * 2 * CHUNK) == 0
    per_w = n_idx // nw
    n_pairs = per_w // (2 * CHUNK)

    mesh = plsc.VectorSubcoreMesh(core_axis_name="c", subcore_axis_name="s")

    @functools.partial(
        pl.kernel,
        mesh=mesh,
        out_type=jax.ShapeDtypeStruct((n_idx, D_PAD), jnp.float32),
        scratch_types=[
            pltpu.VMEM((per_w,), jnp.int32),
            pltpu.VMEM((CHUNK, D_PAD), jnp.float32),
            pltpu.VMEM((CHUNK, D_PAD), jnp.float32),
            pltpu.SemaphoreType.DMA,
            pltpu.SemaphoreType.DMA,
            pltpu.SemaphoreType.DMA,
            pltpu.SemaphoreType.DMA,
        ],
    )
    def run(idx_hbm, table_hbm, out_hbm, idx_v, rows_a, rows_b,
            gsem_a, gsem_b, wsem_a, wsem_b):
        wid = lax.axis_index("s") * nc + lax.axis_index("c")
        base = wid * per_w
        pltpu.sync_copy(idx_hbm.at[pl.ds(pl.multiple_of(base, 16), per_w)],
                        idx_v)

        def fire_gather(c, rows_v, gsem):
            coff = pl.multiple_of(c * CHUNK, 16)
            for (o, n) in _SLICES:
                pltpu.async_copy(
                    table_hbm.at[idx_v.at[pl.ds(coff + o, n)]],
                    rows_v.at[pl.ds(o, n)],
                    gsem)

        def wait_gather(c, rows_v, gsem):
            coff = pl.multiple_of(c * CHUNK, 16)
            for (o, n) in _SLICES:
                pltpu.make_async_copy(
                    table_hbm.at[idx_v.at[pl.ds(coff + o, n)]],
                    rows_v.at[pl.ds(o, n)],
                    gsem).wait()

        def fire_write(c, rows_v, wsem):
            off = pl.multiple_of(base + c * CHUNK, 16)
            pltpu.async_copy(rows_v, out_hbm.at[pl.ds(off, CHUNK)], wsem)

        def wait_write(c, rows_v, wsem):
            off = pl.multiple_of(base + c * CHUNK, 16)
            pltpu.make_async_copy(
                rows_v, out_hbm.at[pl.ds(off, CHUNK)], wsem).wait()

        fire_gather(0, rows_a, gsem_a)

        def body(k, carry):
            c0 = 2 * k
            c1 = c0 + 1
            @pl.when(k > 0)
            def _():
                wait_write(c1 - 2, rows_b, wsem_b)
            fire_gather(c1, rows_b, gsem_b)
            wait_gather(c0, rows_a, gsem_a)
            fire_write(c0, rows_a, wsem_a)
            wait_gather(c1, rows_b, gsem_b)

            @pl.when(k < n_pairs - 1)
            def _():
                wait_write(c0, rows_a, wsem_a)
                fire_gather(c0 + 2, rows_a, gsem_a)
            fire_write(c1, rows_b, wsem_b)
            return carry

        lax.fori_loop(0, n_pairs, body, 0)
        wait_write(2 * n_pairs - 2, rows_a, wsem_a)
        wait_write(2 * n_pairs - 1, rows_b, wsem_b)

    return run


def _pad_transpose_call(n_rows, d):
    blk = 16384

    def body(i_ref, o_ref):
        o_ref[:, :d] = i_ref[...].T
        o_ref[:, d:] = jnp.zeros((blk, D_PAD - d), jnp.float32)

    return pl.pallas_call(
        body,
        grid=(pl.cdiv(n_rows, blk),),
        in_specs=[pl.BlockSpec((d, blk), lambda i: (0, i))],
        out_specs=pl.BlockSpec((blk, D_PAD), lambda i: (i, 0)),
        out_shape=jax.ShapeDtypeStruct((n_rows, D_PAD), jnp.float32),
    )


def kernel(word_ids, embed_table):
    b0, b1 = word_ids.shape
    n_rows, d = embed_table.shape
    n_idx = b0 * b1
    idx1d = word_ids.reshape(-1).astype(jnp.int32)
    table_p = _pad_transpose_call(n_rows, d)(embed_table.T)
    out_pad = _gather_call(n_idx)(idx1d, table_p)
    return out_pad[:, :d].reshape(b0, b1, d)


# speedup vs baseline: 1.0130x; 1.0022x over previous
import functools

import jax
import jax.numpy as jnp
from jax import lax
from jax.experimental import pallas as pl
from jax.experimental.pallas import tpu as pltpu
from jax.experimental.pallas import tpu_sc as plsc

IDX_MINOR = 128
CHUNK = 400
D_PAD = 128
_SLICES = [(0, 80), (80, 80), (160, 80), (240, 80), (320, 80)]


def _gather_call(n_idx):
    info = plsc.get_sparse_core_info()
    nc, ns = info.num_cores, info.num_subcores
    nw = nc * ns
    assert n_idx % (nw * 2 * CHUNK) == 0
    per_w = n_idx // nw
    n_pairs = per_w // (2 * CHUNK)

    mesh = plsc.VectorSubcoreMesh(core_axis_name="c", subcore_axis_name="s")

    @functools.partial(
        pl.kernel,
        mesh=mesh,
        out_type=jax.ShapeDtypeStruct((n_idx, D_PAD), jnp.float32),
        scratch_types=[
            pltpu.VMEM((per_w,), jnp.int32),
            pltpu.VMEM((CHUNK, D_PAD), jnp.float32),
            pltpu.VMEM((CHUNK, D_PAD), jnp.float32),
            pltpu.SemaphoreType.DMA,
            pltpu.SemaphoreType.DMA,
            pltpu.SemaphoreType.DMA,
            pltpu.SemaphoreType.DMA,
        ],
    )
    def run(idx_hbm, table_hbm, out_hbm, idx_v, rows_a, rows_b,
            gsem_a, gsem_b, wsem_a, wsem_b):
        wid = lax.axis_index("s") * nc + lax.axis_index("c")
        base = wid * per_w
        pltpu.sync_copy(idx_hbm.at[pl.ds(pl.multiple_of(base, 16), per_w)],
                        idx_v)

        def fire_gather(c, rows_v, gsem):
            coff = pl.multiple_of(c * CHUNK, 16)
            for (o, n) in _SLICES:
                pltpu.async_copy(
                    table_hbm.at[idx_v.at[pl.ds(coff + o, n)]],
                    rows_v.at[pl.ds(o, n)],
                    gsem)

        def wait_gather(c, rows_v, gsem):
            coff = pl.multiple_of(c * CHUNK, 16)
            for (o, n) in _SLICES:
                pltpu.make_async_copy(
                    table_hbm.at[idx_v.at[pl.ds(coff + o, n)]],
                    rows_v.at[pl.ds(o, n)],
                    gsem).wait()

        def fire_write(c, rows_v, wsem):
            off = pl.multiple_of(base + c * CHUNK, 16)
            pltpu.async_copy(rows_v, out_hbm.at[pl.ds(off, CHUNK)], wsem)

        def wait_write(c, rows_v, wsem):
            off = pl.multiple_of(base + c * CHUNK, 16)
            pltpu.make_async_copy(
                rows_v, out_hbm.at[pl.ds(off, CHUNK)], wsem).wait()

        fire_gather(0, rows_a, gsem_a)

        def body(k, carry):
            c0 = 2 * k
            c1 = c0 + 1
            @pl.when(k > 0)
            def _():
                wait_write(c1 - 2, rows_b, wsem_b)
            fire_gather(c1, rows_b, gsem_b)
            wait_gather(c0, rows_a, gsem_a)
            fire_write(c0, rows_a, wsem_a)
            wait_gather(c1, rows_b, gsem_b)

            @pl.when(k < n_pairs - 1)
            def _():
                wait_write(c0, rows_a, wsem_a)
                fire_gather(c0 + 2, rows_a, gsem_a)
            fire_write(c1, rows_b, wsem_b)
            return carry

        lax.fori_loop(0, n_pairs, body, 0)
        wait_write(2 * n_pairs - 2, rows_a, wsem_a)
        wait_write(2 * n_pairs - 1, rows_b, wsem_b)

    return run


def _pad_transpose_call(n_rows, d):
    blk = 16384

    def body(i_ref, o_ref):
        o_ref[:, :d] = i_ref[...].T
        o_ref[:, d:] = jnp.zeros((blk, D_PAD - d), jnp.float32)

    return pl.pallas_call(
        body,
        grid=(pl.cdiv(n_rows, blk),),
        in_specs=[pl.BlockSpec((d, blk), lambda i: (0, i))],
        out_specs=pl.BlockSpec((blk, D_PAD), lambda i: (i, 0)),
        out_shape=jax.ShapeDtypeStruct((n_rows, D_PAD), jnp.float32),
    )


def kernel(word_ids, embed_table):
    b0, b1 = word_ids.shape
    n_rows, d = embed_table.shape
    n_idx = b0 * b1
    idx1d = word_ids.reshape(-1).astype(jnp.int32)
    table_p = _pad_transpose_call(n_rows, d)(embed_table.T)
    out_pad = _gather_call(n_idx)(idx1d, table_p)
    return out_pad[:, :d].reshape(b0, b1, d)
